# baseline (speedup 1.0000x reference)
import functools

import jax
import jax.numpy as jnp
from jax import lax
from jax.experimental import pallas as pl
from jax.experimental.pallas import tpu as pltpu
from jax.experimental.pallas import tpu_sc as plsc

N_USERS = 25000
N_ITEMS = 25000
D = 64
H = 32
NNZ = 800000
B = 1024
P = 25600
NP = 2 * P
N_SUB = 16
CH = 128
CPS = 28
NSG = 14
CPT = CPS * NSG
NNZP = N_SUB * CPT * CH
SLAB = NP // N_SUB
ZROWS = 32

_mesh = plsc.VectorSubcoreMesh(core_axis_name="c", subcore_axis_name="s")


@functools.partial(
    pl.kernel,
    out_type=(jax.ShapeDtypeStruct((NP, H), jnp.float32),
              jax.ShapeDtypeStruct((NP, H), jnp.float32)),
    mesh=_mesh,
    scratch_types=[
        pltpu.VMEM((CPS, 2, CH), jnp.int32),
        pltpu.VMEM((CPS, CH), jnp.float32),
        pltpu.VMEM((CH, H), jnp.float32),
        pltpu.VMEM((CH, H), jnp.float32),
        pltpu.VMEM((CH, H), jnp.float32),
        pltpu.VMEM((CH, H), jnp.float32),
        pltpu.VMEM((ZROWS, H), jnp.float32),
        pltpu.VMEM_SHARED((NP, H), jnp.float32),
        pltpu.SemaphoreType.DMA((4,)),
        pltpu.SemaphoreType.DMA((4,)),
    ],
    compiler_params=pltpu.CompilerParams(needs_layout_passes=False,
                                         use_tc_tiling_on_sc=False),
)
def _spmm(ego_l, ego_r, packed, vals2, nb_l, nb_r,
          pk_v, vl_v, rb0, rb1, rb2, rb3, zbuf, acc, gsem, ssem):
    c = lax.axis_index("c")
    s = lax.axis_index("s")
    rbufs = (rb0, rb1, rb2, rb3)

    def gather_start(j, k):
        idx = pk_v.at[j, 0]

        @pl.when(c == 0)
        def _():
            pltpu.async_copy(ego_l.at[idx], rbufs[k], gsem.at[k])

        @pl.when(c == 1)
        def _():
            pltpu.async_copy(ego_r.at[idx], rbufs[k], gsem.at[k])

    def gather_wait(k):
        pltpu.make_async_copy(ego_l.at[pk_v.at[0, 0]], rbufs[k],
                              gsem.at[k]).wait()

    def scatter_start(j, k):
        pltpu.async_copy(rbufs[k], acc.at[pk_v.at[j, 1]], ssem.at[k],
                         add=True)

    def scatter_wait(k):
        pltpu.make_async_copy(rbufs[k], acc.at[pk_v.at[0, 1]],
                              ssem.at[k]).wait()

    def scale(j, k):
        rb = rbufs[k]

        def sc16(eb, _):
            vals16 = vl_v[j, pl.ds(eb * 16, 16)]
            rowidx = eb * 16 + lax.iota(jnp.int32, 16)
            for col in range(H):
                cidx = jnp.full((16,), col, jnp.int32)
                x = plsc.load_gather(rb, [rowidx, cidx])
                plsc.store_scatter(rb, [rowidx, cidx], x * vals16)
            return 0
        lax.fori_loop(0, CH // 16, sc16, 0)

    z16 = jnp.zeros((16,), jnp.float32)

    def zfill(i, _):
        zbuf[i, pl.ds(0, 16)] = z16
        zbuf[i, pl.ds(16, 16)] = z16
        return 0
    lax.fori_loop(0, ZROWS, zfill, 0)

    def zslab(k, _):
        pltpu.sync_copy(zbuf, acc.at[pl.ds(s * SLAB + k * ZROWS, ZROWS)])
        return 0
    lax.fori_loop(0, SLAB // ZROWS, zslab, 0)
    plsc.subcore_barrier()

    def supergroup(g, _):
        base = s * CPT + g * CPS
        pltpu.sync_copy(packed.at[pl.ds(base, CPS)], pk_v)
        pltpu.sync_copy(vals2.at[pl.ds(base, CPS)], vl_v)
        gather_start(0, 0)
        gather_start(1, 1)

        def group4(jj, _):
            for k in range(4):
                j = jj * 4 + k
                gather_wait(k)
                k2 = (k + 2) % 4
                if k < 2:
                    @pl.when(jj >= 1)
                    def _():
                        scatter_wait(k2)
                    gather_start(j + 2, k2)
                else:
                    @pl.when(jj * 4 + k + 2 < CPS)
                    def _():
                        scatter_wait(k2)
                        gather_start(j + 2, k2)
                scale(j, k)
                scatter_start(j, k)
            return 0
        lax.fori_loop(0, CPS // 4, group4, 0)
        for k in range(4):
            scatter_wait(k)
        return 0
    lax.fori_loop(0, NSG, supergroup, 0)
    plsc.subcore_barrier()

    @pl.when(c == 0)
    def _():
        pltpu.sync_copy(acc.at[pl.ds(s * SLAB, SLAB)],
                        nb_l.at[pl.ds(s * SLAB, SLAB)])

    @pl.when(c == 1)
    def _():
        pltpu.sync_copy(acc.at[pl.ds(s * SLAB, SLAB)],
                        nb_r.at[pl.ds(s * SLAB, SLAB)])


_BK = 1024


def _dense_body(nbl_ref, nbr_ref, ego_ref, wgc_ref, bgc_ref, wbi_ref, bbi_ref,
                out_ref, outl_ref, outr_ref):
    nb = jnp.concatenate([nbl_ref[...], nbr_ref[...]], axis=1)
    ego = ego_ref[...]
    x = (nb @ wgc_ref[...] + bgc_ref[...]
         + (ego * nb) @ wbi_ref[...] + bbi_ref[...])
    h = jnp.where(x >= 0, x, 0.2 * x)
    norm = jnp.maximum(jnp.sqrt(jnp.sum(h * h, axis=1, keepdims=True)), 1e-12)
    o = h / norm
    out_ref[...] = o
    outl_ref[...] = o[:, :H]
    outr_ref[...] = o[:, H:]


_dense = pl.pallas_call(
    _dense_body,
    grid=(NP // _BK,),
    in_specs=[
        pl.BlockSpec((_BK, H), lambda i: (i, 0)),
        pl.BlockSpec((_BK, H), lambda i: (i, 0)),
        pl.BlockSpec((_BK, D), lambda i: (i, 0)),
        pl.BlockSpec((D, D), lambda i: (0, 0)),
        pl.BlockSpec((1, D), lambda i: (0, 0)),
        pl.BlockSpec((D, D), lambda i: (0, 0)),
        pl.BlockSpec((1, D), lambda i: (0, 0)),
    ],
    out_specs=[
        pl.BlockSpec((_BK, D), lambda i: (i, 0)),
        pl.BlockSpec((_BK, H), lambda i: (i, 0)),
        pl.BlockSpec((_BK, H), lambda i: (i, 0)),
    ],
    out_shape=[
        jax.ShapeDtypeStruct((NP, D), jnp.float32),
        jax.ShapeDtypeStruct((NP, H), jnp.float32),
        jax.ShapeDtypeStruct((NP, H), jnp.float32),
    ],
)

_GPT = B // (2 * N_SUB)


@functools.partial(
    pl.kernel,
    out_type=(jax.ShapeDtypeStruct((4, B, D), jnp.float32),
              jax.ShapeDtypeStruct((4, B, D), jnp.float32),
              jax.ShapeDtypeStruct((4, B, D), jnp.float32)),
    mesh=_mesh,
    scratch_types=[
        pltpu.VMEM((_GPT,), jnp.int32),
        pltpu.VMEM((_GPT, D), jnp.float32),
    ],
    compiler_params=pltpu.CompilerParams(needs_layout_passes=False,
                                         use_tc_tiling_on_sc=False),
)
def _fgather(e0, e1, e2, e3, iu, ip, ing, ou, op, og, idxv, buf):
    c = lax.axis_index("c")
    s = lax.axis_index("s")
    base = (s * 2 + c) * _GPT
    for idx_hbm, out in ((iu, ou), (ip, op), (ing, og)):
        pltpu.sync_copy(idx_hbm.at[pl.ds(base, _GPT)], idxv)
        for k, tab in enumerate((e0, e1, e2, e3)):
            pltpu.sync_copy(tab.at[idxv], buf)
            pltpu.sync_copy(buf, out.at[k, pl.ds(base, _GPT)])


def kernel(adj_indices, adj_values, users, pos_items, neg_items,
           user_emb, item_emb,
           W_gc_0, b_gc_0, W_bi_0, b_bi_0,
           W_gc_1, b_gc_1, W_bi_1, b_bi_1,
           W_gc_2, b_gc_2, W_bi_2, b_bi_2):
    W_gc = (W_gc_0, W_gc_1, W_gc_2)
    b_gc = (b_gc_0, b_gc_1, b_gc_2)
    W_bi = (W_bi_0, W_bi_1, W_bi_2)
    b_bi = (b_bi_0, b_bi_1, b_bi_2)

    row = adj_indices[0].astype(jnp.int32)
    col = adj_indices[1].astype(jnp.int32)
    rowp = row + jnp.where(row >= N_USERS, P - N_USERS, 0).astype(jnp.int32)
    colp = col + jnp.where(col >= N_USERS, P - N_USERS, 0).astype(jnp.int32)
    vals = adj_values.astype(jnp.float32)
    pad = NNZP - NNZ
    colc = jnp.pad(colp, (0, pad)).reshape(-1, CH)
    rowc = jnp.pad(rowp, (0, pad)).reshape(-1, CH)
    packed = jnp.stack([colc, rowc], axis=1)
    vals2 = jnp.pad(vals, (0, pad)).reshape(-1, CH)

    ego = jnp.zeros((NP, D), jnp.float32)
    ego = ego.at[:N_USERS].set(user_emb).at[P:P + N_ITEMS].set(item_emb)
    ego_l = ego[:, :H]
    ego_r = ego[:, H:]

    embs = [ego]
    for k in range(3):
        nb_l, nb_r = _spmm(ego_l, ego_r, packed, vals2)
        ego, ego_l, ego_r = _dense(nb_l, nb_r, ego,
                                   W_gc[k], b_gc[k], W_bi[k], b_bi[k])
        embs.append(ego)

    iu = users.astype(jnp.int32)
    ip = pos_items.astype(jnp.int32) + P
    ig = neg_items.astype(jnp.int32) + P
    ou, opos, oneg = _fgather(embs[0], embs[1], embs[2], embs[3], iu, ip, ig)
    u_g = ou.transpose(1, 0, 2).reshape(B, 4 * D)
    pos_g = opos.transpose(1, 0, 2).reshape(B, 4 * D)
    neg_g = oneg.transpose(1, 0, 2).reshape(B, 4 * D)
    return (u_g, pos_g, neg_g)


# speedup vs baseline: 5.1950x; 5.1950x over previous
import functools

import jax
import jax.numpy as jnp
from jax import lax
from jax.experimental import pallas as pl
from jax.experimental.pallas import tpu as pltpu
from jax.experimental.pallas import tpu_sc as plsc

N_USERS = 25000
N_ITEMS = 25000
D = 64
H = 32
NNZ = 800000
B = 1024
P = 25600
NP = 2 * P
N_SUB = 16
CH = 128
CPS = 28
NSG = 14
CPT = CPS * NSG
NNZP = N_SUB * CPT * CH
SLAB = NP // N_SUB
ZROWS = 32

_mesh = plsc.VectorSubcoreMesh(core_axis_name="c", subcore_axis_name="s")


@functools.partial(
    pl.kernel,
    out_type=(jax.ShapeDtypeStruct((NP, H), jnp.float32),
              jax.ShapeDtypeStruct((NP, H), jnp.float32)),
    mesh=_mesh,
    scratch_types=[
        pltpu.VMEM((CPS, 2, CH), jnp.int32),
        pltpu.VMEM((CPS, CH), jnp.float32),
        pltpu.VMEM((CH, H), jnp.float32),
        pltpu.VMEM((CH, H), jnp.float32),
        pltpu.VMEM((CH, H), jnp.float32),
        pltpu.VMEM((CH, H), jnp.float32),
        pltpu.VMEM((ZROWS, H), jnp.float32),
        pltpu.VMEM_SHARED((NP, H), jnp.float32),
        pltpu.SemaphoreType.DMA((4,)),
        pltpu.SemaphoreType.DMA((4,)),
    ],
    compiler_params=pltpu.CompilerParams(needs_layout_passes=False,
                                         use_tc_tiling_on_sc=False),
)
def _spmm(ego_l, ego_r, packed, vals2, nb_l, nb_r,
          pk_v, vl_v, rb0, rb1, rb2, rb3, zbuf, acc, gsem, ssem):
    c = lax.axis_index("c")
    s = lax.axis_index("s")
    rbufs = (rb0, rb1, rb2, rb3)

    def gather_start(j, k):
        idx = pk_v.at[j, 0]

        @pl.when(c == 0)
        def _():
            pltpu.async_copy(ego_l.at[idx], rbufs[k], gsem.at[k])

        @pl.when(c == 1)
        def _():
            pltpu.async_copy(ego_r.at[idx], rbufs[k], gsem.at[k])

    def gather_wait(k):
        pltpu.make_async_copy(ego_l.at[pk_v.at[0, 0]], rbufs[k],
                              gsem.at[k]).wait()

    def scatter_start(j, k):
        pltpu.async_copy(rbufs[k], acc.at[pk_v.at[j, 1]], ssem.at[k],
                         add=True)

    def scatter_wait(k):
        pltpu.make_async_copy(rbufs[k], acc.at[pk_v.at[0, 1]],
                              ssem.at[k]).wait()

    def scale(j, k):
        rb = rbufs[k]

        def sc16(eb, _):
            vals16 = vl_v[j, pl.ds(eb * 16, 16)]
            for ek in range(16):
                vv = vals16.at[jnp.full((16,), ek, jnp.int32)].get(
                    mode="promise_in_bounds")
                e = eb * 16 + ek
                rb[e, pl.ds(0, 16)] = rb[e, pl.ds(0, 16)] * vv
                rb[e, pl.ds(16, 16)] = rb[e, pl.ds(16, 16)] * vv
            return 0
        lax.fori_loop(0, CH // 16, sc16, 0)

    z16 = jnp.zeros((16,), jnp.float32)

    def zfill(i, _):
        zbuf[i, pl.ds(0, 16)] = z16
        zbuf[i, pl.ds(16, 16)] = z16
        return 0
    lax.fori_loop(0, ZROWS, zfill, 0)

    def zslab(k, _):
        pltpu.sync_copy(zbuf, acc.at[pl.ds(s * SLAB + k * ZROWS, ZROWS)])
        return 0
    lax.fori_loop(0, SLAB // ZROWS, zslab, 0)
    plsc.subcore_barrier()

    def supergroup(g, _):
        base = s * CPT + g * CPS
        pltpu.sync_copy(packed.at[pl.ds(base, CPS)], pk_v)
        pltpu.sync_copy(vals2.at[pl.ds(base, CPS)], vl_v)
        gather_start(0, 0)
        gather_start(1, 1)

        def group4(jj, _):
            for k in range(4):
                j = jj * 4 + k
                gather_wait(k)
                k2 = (k + 2) % 4
                if k < 2:
                    @pl.when(jj >= 1)
                    def _():
                        scatter_wait(k2)
                    gather_start(j + 2, k2)
                else:
                    @pl.when(jj * 4 + k + 2 < CPS)
                    def _():
                        scatter_wait(k2)
                        gather_start(j + 2, k2)
                scale(j, k)
                scatter_start(j, k)
            return 0
        lax.fori_loop(0, CPS // 4, group4, 0)
        for k in range(4):
            scatter_wait(k)
        return 0
    lax.fori_loop(0, NSG, supergroup, 0)
    plsc.subcore_barrier()

    @pl.when(c == 0)
    def _():
        pltpu.sync_copy(acc.at[pl.ds(s * SLAB, SLAB)],
                        nb_l.at[pl.ds(s * SLAB, SLAB)])

    @pl.when(c == 1)
    def _():
        pltpu.sync_copy(acc.at[pl.ds(s * SLAB, SLAB)],
                        nb_r.at[pl.ds(s * SLAB, SLAB)])


_BK = 6400


def _dense_body(nbl_ref, nbr_ref, ego_ref, wgc_ref, bgc_ref, wbi_ref, bbi_ref,
                out_ref, outl_ref, outr_ref):
    nb = jnp.concatenate([nbl_ref[...], nbr_ref[...]], axis=1)
    ego = ego_ref[...]
    x = (nb @ wgc_ref[...] + bgc_ref[...]
         + (ego * nb) @ wbi_ref[...] + bbi_ref[...])
    h = jnp.where(x >= 0, x, 0.2 * x)
    norm = jnp.maximum(jnp.sqrt(jnp.sum(h * h, axis=1, keepdims=True)), 1e-12)
    o = h / norm
    out_ref[...] = o
    outl_ref[...] = o[:, :H]
    outr_ref[...] = o[:, H:]


_dense = pl.pallas_call(
    _dense_body,
    grid=(NP // _BK,),
    in_specs=[
        pl.BlockSpec((_BK, H), lambda i: (i, 0)),
        pl.BlockSpec((_BK, H), lambda i: (i, 0)),
        pl.BlockSpec((_BK, D), lambda i: (i, 0)),
        pl.BlockSpec((D, D), lambda i: (0, 0)),
        pl.BlockSpec((1, D), lambda i: (0, 0)),
        pl.BlockSpec((D, D), lambda i: (0, 0)),
        pl.BlockSpec((1, D), lambda i: (0, 0)),
    ],
    out_specs=[
        pl.BlockSpec((_BK, D), lambda i: (i, 0)),
        pl.BlockSpec((_BK, H), lambda i: (i, 0)),
        pl.BlockSpec((_BK, H), lambda i: (i, 0)),
    ],
    out_shape=[
        jax.ShapeDtypeStruct((NP, D), jnp.float32),
        jax.ShapeDtypeStruct((NP, H), jnp.float32),
        jax.ShapeDtypeStruct((NP, H), jnp.float32),
    ],
)

_GPT = B // (2 * N_SUB)


@functools.partial(
    pl.kernel,
    out_type=(jax.ShapeDtypeStruct((4, B, D), jnp.float32),
              jax.ShapeDtypeStruct((4, B, D), jnp.float32),
              jax.ShapeDtypeStruct((4, B, D), jnp.float32)),
    mesh=_mesh,
    scratch_types=[
        pltpu.VMEM((_GPT,), jnp.int32),
        pltpu.VMEM((_GPT, D), jnp.float32),
    ],
    compiler_params=pltpu.CompilerParams(needs_layout_passes=False,
                                         use_tc_tiling_on_sc=False),
)
def _fgather(e0, e1, e2, e3, iu, ip, ing, ou, op, og, idxv, buf):
    c = lax.axis_index("c")
    s = lax.axis_index("s")
    base = (s * 2 + c) * _GPT
    for idx_hbm, out in ((iu, ou), (ip, op), (ing, og)):
        pltpu.sync_copy(idx_hbm.at[pl.ds(base, _GPT)], idxv)
        for k, tab in enumerate((e0, e1, e2, e3)):
            pltpu.sync_copy(tab.at[idxv], buf)
            pltpu.sync_copy(buf, out.at[k, pl.ds(base, _GPT)])


def kernel(adj_indices, adj_values, users, pos_items, neg_items,
           user_emb, item_emb,
           W_gc_0, b_gc_0, W_bi_0, b_bi_0,
           W_gc_1, b_gc_1, W_bi_1, b_bi_1,
           W_gc_2, b_gc_2, W_bi_2, b_bi_2):
    W_gc = (W_gc_0, W_gc_1, W_gc_2)
    b_gc = (b_gc_0, b_gc_1, b_gc_2)
    W_bi = (W_bi_0, W_bi_1, W_bi_2)
    b_bi = (b_bi_0, b_bi_1, b_bi_2)

    row = adj_indices[0].astype(jnp.int32)
    col = adj_indices[1].astype(jnp.int32)
    rowp = row + jnp.where(row >= N_USERS, P - N_USERS, 0).astype(jnp.int32)
    colp = col + jnp.where(col >= N_USERS, P - N_USERS, 0).astype(jnp.int32)
    vals = adj_values.astype(jnp.float32)
    pad = NNZP - NNZ
    colc = jnp.pad(colp, (0, pad)).reshape(-1, CH)
    rowc = jnp.pad(rowp, (0, pad)).reshape(-1, CH)
    packed = jnp.stack([colc, rowc], axis=1)
    vals2 = jnp.pad(vals, (0, pad)).reshape(-1, CH)

    ego = jnp.zeros((NP, D), jnp.float32)
    ego = ego.at[:N_USERS].set(user_emb).at[P:P + N_ITEMS].set(item_emb)
    ego_l = ego[:, :H]
    ego_r = ego[:, H:]

    embs = [ego]
    for k in range(3):
        nb_l, nb_r = _spmm(ego_l, ego_r, packed, vals2)
        ego, ego_l, ego_r = _dense(nb_l, nb_r, ego,
                                   W_gc[k], b_gc[k], W_bi[k], b_bi[k])
        embs.append(ego)

    iu = users.astype(jnp.int32)
    ip = pos_items.astype(jnp.int32) + P
    ig = neg_items.astype(jnp.int32) + P
    ou, opos, oneg = _fgather(embs[0], embs[1], embs[2], embs[3], iu, ip, ig)
    u_g = ou.transpose(1, 0, 2).reshape(B, 4 * D)
    pos_g = opos.transpose(1, 0, 2).reshape(B, 4 * D)
    neg_g = oneg.transpose(1, 0, 2).reshape(B, 4 * D)
    return (u_g, pos_g, neg_g)
